# Initial kernel scaffold; baseline (speedup 1.0000x reference)
import jax
import jax.numpy as jnp
from jax.experimental import pallas as pl


def kernel(x, W_u1, b_u1, W_u2, b_u2, W_router, We1, be1, We2, be2):
    raise NotImplementedError("write your pallas kernel here")


# speedup vs baseline: 2.7233x; 2.7233x over previous
import functools

import jax
import jax.numpy as jnp
from jax import lax
from jax.experimental import pallas as pl
from jax.experimental.pallas import tpu as pltpu

N = 4096
D = 768
E = 8
K = 2
U = 16
H = 4 * D

BM = 512


def _gelu_exact(t):
    return 0.5 * t * (1.0 + lax.erf(t * 0.7071067811865476))


def _router_body(x_ref, wu1_ref, bu1_ref, wu2_ref, bu2_ref, wrt_ref,
                 comb_ref, aux_ref):
    x = x_ref[...]
    h = lax.dot_general(x, wu1_ref[...], (((1,), (1,)), ((), ())),
                        preferred_element_type=jnp.float32)
    h = _gelu_exact(h + bu1_ref[...])
    u = jnp.sum(h * wu2_ref[...], axis=-1, keepdims=True)
    u = u + bu2_ref[0, 0]
    u = jnp.maximum(u, 0.0) + jnp.log1p(jnp.exp(-jnp.abs(u)))
    un = u / (jnp.mean(u) + 1e-8)
    logits = lax.dot_general(x, wrt_ref[:D, :], (((1,), (0,)), ((), ())),
                             preferred_element_type=jnp.float32)
    logits = logits + un * wrt_ref[D:D + 1, :]
    m = jnp.max(logits, axis=-1, keepdims=True)
    ex = jnp.exp(logits - m)
    se = jnp.sum(ex, axis=-1, keepdims=True)
    p = ex / se
    iota = lax.broadcasted_iota(jnp.int32, (N, E), 1)
    m1 = jnp.max(p, axis=-1, keepdims=True)
    i1 = jnp.min(jnp.where(p == m1, iota, E), axis=-1, keepdims=True)
    p2 = jnp.where(iota == i1, -jnp.inf, p)
    m2 = jnp.max(p2, axis=-1, keepdims=True)
    i2 = jnp.min(jnp.where(p2 == m2, iota, E), axis=-1, keepdims=True)
    s = m1 + m2
    sel1 = (iota == i1).astype(jnp.float32)
    sel2 = (iota == i2).astype(jnp.float32)
    comb_ref[...] = sel1 * (m1 / s) + sel2 * (m2 / s)
    usage = jnp.mean(p, axis=0, keepdims=True)
    selection = jnp.mean(sel1 + sel2, axis=0, keepdims=True) / K
    lb = E * jnp.sum(usage * selection)
    lz = m + jnp.log(se)
    z = jnp.mean(lz * lz)
    aux_ref[0, 0] = 0.01 * lb + 0.01 * z


def _expert_body(x_ref, comb_ref, w1_ref, b1_ref, w2_ref, b2_ref, out_ref):
    e = pl.program_id(1)
    h = lax.dot_general(x_ref[...], w1_ref[0], (((1,), (1,)), ((), ())),
                        preferred_element_type=jnp.float32)
    h = _gelu_exact(h + b1_ref[0])
    o = lax.dot_general(h.astype(jnp.bfloat16), w2_ref[0],
                        (((1,), (1,)), ((), ())),
                        preferred_element_type=jnp.float32)
    o = o + b2_ref[0]
    eiota = lax.broadcasted_iota(jnp.int32, (1, E), 1)
    col = jnp.sum(jnp.where(eiota == e, comb_ref[...], 0.0), axis=-1,
                  keepdims=True)

    @pl.when(e == 0)
    def _():
        out_ref[...] = jnp.zeros_like(out_ref)

    out_ref[...] += col * o


@jax.jit
def kernel(x, W_u1, b_u1, W_u2, b_u2, W_router, We1, be1, We2, be2):
    comb, aux = pl.pallas_call(
        _router_body,
        out_shape=(jax.ShapeDtypeStruct((N, E), jnp.float32),
                   jax.ShapeDtypeStruct((1, 1), jnp.float32)),
        in_specs=[
            pl.BlockSpec((N, D), lambda: (0, 0)),
            pl.BlockSpec((U, D), lambda: (0, 0)),
            pl.BlockSpec((1, U), lambda: (0, 0)),
            pl.BlockSpec((1, U), lambda: (0, 0)),
            pl.BlockSpec(memory_space=pltpu.SMEM),
            pl.BlockSpec((D + 1, E), lambda: (0, 0)),
        ],
        out_specs=(pl.BlockSpec((N, E), lambda: (0, 0)),
                   pl.BlockSpec(memory_space=pltpu.SMEM)),
    )(x, W_u1, b_u1.reshape(1, U), W_u2, b_u2.reshape(1, 1), W_router.T)

    xb = x.astype(jnp.bfloat16)
    We1b = We1.astype(jnp.bfloat16)
    We2b = We2.astype(jnp.bfloat16)
    be1r = be1.reshape(E, 1, H)
    be2r = be2.reshape(E, 1, D)

    out = pl.pallas_call(
        _expert_body,
        grid=(N // BM, E),
        out_shape=jax.ShapeDtypeStruct((N, D), jnp.float32),
        in_specs=[
            pl.BlockSpec((BM, D), lambda n, e: (n, 0)),
            pl.BlockSpec((BM, E), lambda n, e: (n, 0)),
            pl.BlockSpec((1, H, D), lambda n, e: (e, 0, 0)),
            pl.BlockSpec((1, 1, H), lambda n, e: (e, 0, 0)),
            pl.BlockSpec((1, D, H), lambda n, e: (e, 0, 0)),
            pl.BlockSpec((1, 1, D), lambda n, e: (e, 0, 0)),
        ],
        out_specs=pl.BlockSpec((BM, D), lambda n, e: (n, 0)),
        compiler_params=pltpu.CompilerParams(
            dimension_semantics=("parallel", "arbitrary")),
    )(xb, comb, We1b, be1r, We2b, be2r)

    return out, aux.reshape(())
